# baseline (speedup 1.0000x reference)
import functools

import jax
import jax.numpy as jnp
from jax import lax
from jax.experimental import pallas as pl
from jax.experimental.pallas import tpu as pltpu
from jax.experimental.pallas import tpu_sc as plsc

NC = 2
NS = 16
NW = NC * NS
L = 16
SUB = 128
NB = 3


def _deg_body(n, kc, dst_hbm, ew_hbm, out_hbm, dst_v, ew_v, deg_v):
  c = lax.axis_index("c")
  s = lax.axis_index("s")
  wid = s * NC + c
  pltpu.sync_copy(dst_hbm.at[wid], dst_v)
  pltpu.sync_copy(ew_hbm.at[wid], ew_v)

  zero = jnp.zeros((L,), jnp.float32)

  def zbody(i, _):
    deg_v[pl.ds(i * L, L)] = zero
    return 0

  lax.fori_loop(0, n // L, zbody, 0)

  def ebody(i, _):
    j = i // (SUB // L)
    g = i % (SUB // L)
    idx = dst_v[j, pl.ds(g * L, L)]
    w = ew_v[j, pl.ds(g * L, L)]
    plsc.addupdate_scatter(deg_v, [idx], w)
    return 0

  lax.fori_loop(0, kc * (SUB // L), ebody, 0)
  pltpu.sync_copy(deg_v, out_hbm.at[wid])


def _deg_call(n, kc, dst3, ew3):
  mesh = plsc.VectorSubcoreMesh(core_axis_name="c", subcore_axis_name="s")
  return pl.kernel(
      functools.partial(_deg_body, n, kc),
      out_type=jax.ShapeDtypeStruct((NW, n), jnp.float32),
      mesh=mesh,
      compiler_params=pltpu.CompilerParams(needs_layout_passes=False),
      scratch_types=[
          pltpu.VMEM((kc, SUB), jnp.int32),
          pltpu.VMEM((kc, SUB), jnp.float32),
          pltpu.VMEM((n,), jnp.float32),
      ],
  )(dst3, ew3)


def _msg_body(n, d, kc, h_hbm, src_hbm, dst_hbm, ew_hbm, out_hbm,
              src_r, dst_r, ew_r, rows0, rows1, rows2,
              g0, g1, g2, s0, s1, s2, acc_sp):
  c = lax.axis_index("c")
  s = lax.axis_index("s")
  wid = s * NC + c
  nv = d // L
  rows = (rows0, rows1, rows2)
  gsem = (g0, g1, g2)
  ssem = (s0, s1, s2)

  stride = (n // NS) // 8 * 8
  nwin = -(-(n - stride * (NS - 1)) // SUB)
  base = pl.multiple_of(s * stride, 8)

  zero = jnp.zeros((L,), jnp.float32)

  def zb(r, _):
    for k in range(nv):
      rows0[r, pl.ds(k * L, L)] = zero
    return 0

  lax.fori_loop(0, SUB, zb, 0)
  for t in range(nwin):
    pltpu.sync_copy(rows0, acc_sp.at[pl.ds(base + t * SUB, SUB)])
  plsc.subcore_barrier()

  lane_ids = [jnp.full((L,), l, jnp.int32) for l in range(L)]

  def load_idx(b, j):
    pltpu.sync_copy(src_hbm.at[wid, j], src_r.at[b])
    pltpu.sync_copy(dst_hbm.at[wid, j], dst_r.at[b])
    pltpu.sync_copy(ew_hbm.at[wid, j], ew_r.at[b])

  for b in range(NB):
    load_idx(b, b)
    pltpu.async_copy(h_hbm.at[src_r.at[b]], rows[b], gsem[b])

  def rounds(p, _):
    for b in range(NB):
      j = p * NB + b
      rv = rows[b]
      pltpu.make_async_copy(h_hbm.at[src_r.at[b]], rv, gsem[b]).wait()

      def grp(g, _):
        ewv = ew_r[b, pl.ds(g * L, L)]
        for l in range(L):
          w = jnp.take_along_axis(ewv, lane_ids[l], axis=0)
          e = g * L + l
          for k in range(nv):
            sl = pl.ds(k * L, L)
            rv[e, sl] = rv[e, sl] * w
        return 0

      lax.fori_loop(0, SUB // L, grp, 0)

      pltpu.async_copy(rv, acc_sp.at[dst_r.at[b]], ssem[b], add=True)
      pltpu.make_async_copy(rv, acc_sp.at[dst_r.at[b]], ssem[b]).wait()

      @pl.when(j + NB < kc)
      def _():
        load_idx(b, j + NB)
        pltpu.async_copy(h_hbm.at[src_r.at[b]], rv, gsem[b])

    return 0

  lax.fori_loop(0, kc // NB, rounds, 0)
  plsc.subcore_barrier()
  pltpu.sync_copy(acc_sp.at[pl.ds(base, nwin * SUB)],
                  out_hbm.at[c, pl.ds(base, nwin * SUB)])


def _msg_call(n, d, kc, h, src3, dst3, ew3):
  mesh = plsc.VectorSubcoreMesh(core_axis_name="c", subcore_axis_name="s")
  return pl.kernel(
      functools.partial(_msg_body, n, d, kc),
      out_type=jax.ShapeDtypeStruct((NC, n, d), jnp.float32),
      mesh=mesh,
      compiler_params=pltpu.CompilerParams(needs_layout_passes=False),
      scratch_types=[
          pltpu.VMEM((NB, SUB), jnp.int32),
          pltpu.VMEM((NB, SUB), jnp.int32),
          pltpu.VMEM((NB, SUB), jnp.float32),
          pltpu.VMEM((SUB, d), jnp.float32),
          pltpu.VMEM((SUB, d), jnp.float32),
          pltpu.VMEM((SUB, d), jnp.float32),
          pltpu.SemaphoreType.DMA,
          pltpu.SemaphoreType.DMA,
          pltpu.SemaphoreType.DMA,
          pltpu.SemaphoreType.DMA,
          pltpu.SemaphoreType.DMA,
          pltpu.SemaphoreType.DMA,
          pltpu.VMEM_SHARED((n, d), jnp.float32),
      ],
  )(h, src3, dst3, ew3)


def _stage0(degp):
  n = degp.shape[1]

  def body(degp_ref, out_ref):
    deg = jnp.sum(degp_ref[...], axis=0) + 1.0
    pos = deg > 0
    dis = jnp.where(pos, lax.rsqrt(jnp.where(pos, deg, 1.0)), 0.0)
    out_ref[...] = dis[:, None]

  return pl.pallas_call(
      body,
      out_shape=jax.ShapeDtypeStruct((n, 1), jnp.float32),
  )(degp)


def _stage1(x, W1, dis2, blk):
  n, di = x.shape
  dh = W1.shape[1]

  def body(x_ref, w_ref, dis_ref, out_ref):
    h = jnp.dot(x_ref[...], w_ref[...], precision=lax.Precision.HIGHEST,
                preferred_element_type=jnp.float32)
    out_ref[...] = h * dis_ref[...]

  return pl.pallas_call(
      body,
      grid=(n // blk,),
      in_specs=[
          pl.BlockSpec((blk, di), lambda i: (i, 0)),
          pl.BlockSpec((di, dh), lambda i: (0, 0)),
          pl.BlockSpec((blk, 1), lambda i: (i, 0)),
      ],
      out_specs=pl.BlockSpec((blk, dh), lambda i: (i, 0)),
      out_shape=jax.ShapeDtypeStruct((n, dh), jnp.float32),
  )(x, W1, dis2)


def _stage2(s1, hp1, dis2, b1, W2, blk):
  n, dh = hp1.shape
  do = W2.shape[1]

  def body(s_ref, hp_ref, dis_ref, b_ref, w_ref, out_ref):
    dis = dis_ref[...]
    z = dis * (s_ref[0] + s_ref[1] + hp_ref[...]) + b_ref[...]
    a = jnp.where(z >= 0, z, 0.01 * z)
    h2 = jnp.dot(a, w_ref[...], precision=lax.Precision.HIGHEST,
                 preferred_element_type=jnp.float32)
    out_ref[...] = h2 * dis

  return pl.pallas_call(
      body,
      grid=(n // blk,),
      in_specs=[
          pl.BlockSpec((NC, blk, dh), lambda i: (0, i, 0)),
          pl.BlockSpec((blk, dh), lambda i: (i, 0)),
          pl.BlockSpec((blk, 1), lambda i: (i, 0)),
          pl.BlockSpec((1, dh), lambda i: (0, 0)),
          pl.BlockSpec((dh, do), lambda i: (0, 0)),
      ],
      out_specs=pl.BlockSpec((blk, do), lambda i: (i, 0)),
      out_shape=jax.ShapeDtypeStruct((n, do), jnp.float32),
  )(s1, hp1, dis2, b1, W2)


def _stage3(s2, hp2, dis2, b2, gamma, beta, blk):
  n, do = hp2.shape

  def body(s_ref, hp_ref, dis_ref, b_ref, g_ref, be_ref, out_ref):
    z = dis_ref[...] * (s_ref[0] + s_ref[1] + hp_ref[...]) + b_ref[...]
    mu = jnp.mean(z, axis=-1, keepdims=True)
    var = jnp.mean((z - mu) ** 2, axis=-1, keepdims=True)
    out_ref[...] = (z - mu) * lax.rsqrt(var + 1e-5) * g_ref[...] + be_ref[...]

  return pl.pallas_call(
      body,
      grid=(n // blk,),
      in_specs=[
          pl.BlockSpec((NC, blk, do), lambda i: (0, i, 0)),
          pl.BlockSpec((blk, do), lambda i: (i, 0)),
          pl.BlockSpec((blk, 1), lambda i: (i, 0)),
          pl.BlockSpec((1, do), lambda i: (0, 0)),
          pl.BlockSpec((1, do), lambda i: (0, 0)),
          pl.BlockSpec((1, do), lambda i: (0, 0)),
      ],
      out_specs=pl.BlockSpec((blk, do), lambda i: (i, 0)),
      out_shape=jax.ShapeDtypeStruct((n, do), jnp.float32),
  )(s2, hp2, dis2, b2, gamma, beta)


def kernel(x, edge_index, edge_weight, W1, b1, W2, b2, gamma, beta):
  n = x.shape[0]
  e = edge_weight.shape[0]
  src, dst = edge_index[0], edge_index[1]

  kc = -(-e // (NW * SUB))
  kc = -(-kc // NB) * NB
  pad = NW * kc * SUB - e
  src3 = jnp.pad(src, (0, pad)).reshape(NW, kc, SUB)
  dst3 = jnp.pad(dst, (0, pad)).reshape(NW, kc, SUB)
  ew3 = jnp.pad(edge_weight, (0, pad)).reshape(NW, kc, SUB)

  blk = 1000

  degp = _deg_call(n, kc, dst3, ew3)
  dis2 = _stage0(degp)
  hp1 = _stage1(x, W1, dis2, blk)
  s1 = _msg_call(n, hp1.shape[1], kc, hp1, src3, dst3, ew3)
  hp2 = _stage2(s1, hp1, dis2, b1.reshape(1, -1), W2, blk)
  s2 = _msg_call(n, hp2.shape[1], kc, hp2, src3, dst3, ew3)
  return _stage3(s2, hp2, dis2, b2.reshape(1, -1),
                 gamma.reshape(1, -1), beta.reshape(1, -1), blk)


# speedup vs baseline: 1.1745x; 1.1745x over previous
import functools

import jax
import jax.numpy as jnp
from jax import lax
from jax.experimental import pallas as pl
from jax.experimental.pallas import tpu as pltpu
from jax.experimental.pallas import tpu_sc as plsc

NC = 2
NS = 16
NW = NC * NS
L = 16
SUB = 128
NB = 2
PB = 8


def _deg_body(n, kc, dst_hbm, ew_hbm, out_hbm, dst_v, ew_v, deg_v):
  c = lax.axis_index("c")
  s = lax.axis_index("s")
  wid = s * NC + c
  pltpu.sync_copy(dst_hbm.at[wid], dst_v)
  pltpu.sync_copy(ew_hbm.at[wid], ew_v)

  zero = jnp.zeros((L,), jnp.float32)

  def zbody(i, _):
    deg_v[pl.ds(i * L, L)] = zero
    return 0

  lax.fori_loop(0, n // L, zbody, 0)

  def ebody(i, _):
    j = i // (SUB // L)
    g = i % (SUB // L)
    idx = dst_v[j, pl.ds(g * L, L)]
    w = ew_v[j, pl.ds(g * L, L)]
    plsc.addupdate_scatter(deg_v, [idx], w)
    return 0

  lax.fori_loop(0, kc * (SUB // L), ebody, 0)
  pltpu.sync_copy(deg_v, out_hbm.at[wid])


def _deg_call(n, kc, dst3, ew3):
  mesh = plsc.VectorSubcoreMesh(core_axis_name="c", subcore_axis_name="s")
  return pl.kernel(
      functools.partial(_deg_body, n, kc),
      out_type=jax.ShapeDtypeStruct((NW, n), jnp.float32),
      mesh=mesh,
      compiler_params=pltpu.CompilerParams(needs_layout_passes=False),
      scratch_types=[
          pltpu.VMEM((kc, SUB), jnp.int32),
          pltpu.VMEM((kc, SUB), jnp.float32),
          pltpu.VMEM((n,), jnp.float32),
      ],
  )(dst3, ew3)


def _msg_body(n, d, kc, h_hbm, src_hbm, dst_hbm, ew_hbm, out_hbm,
              src_p, dst_p, ew_p, rows0, rows1,
              g0, g1, s0, s1, isem, acc_sp):
  c = lax.axis_index("c")
  s = lax.axis_index("s")
  wid = s * NC + c
  nv = d // L
  nm = kc // PB
  rows = (rows0, rows1)
  gsem = (g0, g1)
  ssem = (s0, s1)

  stride = (n // NS) // 8 * 8
  nwin = -(-(n - stride * (NS - 1)) // SUB)
  base = pl.multiple_of(s * stride, 8)

  zero = jnp.zeros((L,), jnp.float32)

  def zb(r, _):
    for k in range(nv):
      rows0[r, pl.ds(k * L, L)] = zero
    return 0

  lax.fori_loop(0, SUB, zb, 0)
  for t in range(nwin):
    pltpu.sync_copy(rows0, acc_sp.at[pl.ds(base + t * SUB, SUB)])
  plsc.subcore_barrier()

  lane_ids = [jnp.full((L,), l, jnp.int32) for l in range(L)]
  planes = ((src_hbm, src_p), (dst_hbm, dst_p), (ew_hbm, ew_p))

  def refresh_plane(pidx, m):
    for hbm, pbuf in planes:
      pltpu.async_copy(hbm.at[wid, pl.ds(m * PB, PB)], pbuf.at[pidx], isem)

  def wait_plane(pidx, m):
    for hbm, pbuf in planes:
      pltpu.make_async_copy(hbm.at[wid, pl.ds(m * PB, PB)], pbuf.at[pidx],
                            isem).wait()

  for hbm, pbuf in planes:
    pltpu.sync_copy(hbm.at[wid, pl.ds(0, PB)], pbuf.at[0])
  if nm > 1:
    refresh_plane(1, 1)
  for b in range(NB):
    pltpu.async_copy(h_hbm.at[src_p.at[0, b]], rows[b], gsem[b])

  def macro(m, _):
    pm = m % 2
    pmn = 1 - pm
    for b8 in range(PB):
      j = m * PB + b8
      b = b8 % 2
      rv = rows[b]

      if b8 == 0:

        @pl.when(jnp.logical_and(m >= 1, m + 1 < nm))
        def _():
          refresh_plane(pmn, m + 1)

      pltpu.make_async_copy(h_hbm.at[src_p.at[pm, b8]], rv, gsem[b]).wait()

      def grp(g, _):
        ewv = ew_p[pm, b8, pl.ds(g * L, L)]
        for l in range(L):
          w = jnp.take_along_axis(ewv, lane_ids[l], axis=0)
          e = g * L + l
          for k in range(nv):
            sl = pl.ds(k * L, L)
            rv[e, sl] = rv[e, sl] * w
        return 0

      lax.fori_loop(0, SUB // L, grp, 0)

      pltpu.async_copy(rv, acc_sp.at[dst_p.at[pm, b8]], ssem[b], add=True)
      pltpu.make_async_copy(rv, acc_sp.at[dst_p.at[pm, b8]], ssem[b]).wait()

      if b8 == PB - NB:

        @pl.when(m + 1 < nm)
        def _():
          wait_plane(pmn, m + 1)

      if b8 < PB - NB:
        nsrc = src_p.at[pm, b8 + NB]
      else:
        nsrc = src_p.at[pmn, b8 + NB - PB]

      @pl.when(j + NB < kc)
      def _():
        pltpu.async_copy(h_hbm.at[nsrc], rv, gsem[b])

    return 0

  lax.fori_loop(0, nm, macro, 0)
  plsc.subcore_barrier()
  pltpu.sync_copy(acc_sp.at[pl.ds(base, nwin * SUB)],
                  out_hbm.at[c, pl.ds(base, nwin * SUB)])


def _msg_call(n, d, kc, h, src3, dst3, ew3):
  mesh = plsc.VectorSubcoreMesh(core_axis_name="c", subcore_axis_name="s")
  return pl.kernel(
      functools.partial(_msg_body, n, d, kc),
      out_type=jax.ShapeDtypeStruct((NC, n, d), jnp.float32),
      mesh=mesh,
      compiler_params=pltpu.CompilerParams(needs_layout_passes=False),
      scratch_types=[
          pltpu.VMEM((2, PB, SUB), jnp.int32),
          pltpu.VMEM((2, PB, SUB), jnp.int32),
          pltpu.VMEM((2, PB, SUB), jnp.float32),
          pltpu.VMEM((SUB, d), jnp.float32),
          pltpu.VMEM((SUB, d), jnp.float32),
          pltpu.SemaphoreType.DMA,
          pltpu.SemaphoreType.DMA,
          pltpu.SemaphoreType.DMA,
          pltpu.SemaphoreType.DMA,
          pltpu.SemaphoreType.DMA,
          pltpu.VMEM_SHARED((n, d), jnp.float32),
      ],
  )(h, src3, dst3, ew3)


def _stage0(degp):
  n = degp.shape[1]

  def body(degp_ref, out_ref):
    deg = jnp.sum(degp_ref[...], axis=0) + 1.0
    pos = deg > 0
    dis = jnp.where(pos, lax.rsqrt(jnp.where(pos, deg, 1.0)), 0.0)
    out_ref[...] = dis[:, None]

  return pl.pallas_call(
      body,
      out_shape=jax.ShapeDtypeStruct((n, 1), jnp.float32),
  )(degp)


def _stage1(x, W1, dis2, blk):
  n, di = x.shape
  dh = W1.shape[1]

  def body(x_ref, w_ref, dis_ref, out_ref):
    h = jnp.dot(x_ref[...], w_ref[...], precision=lax.Precision.HIGHEST,
                preferred_element_type=jnp.float32)
    out_ref[...] = h * dis_ref[...]

  return pl.pallas_call(
      body,
      grid=(n // blk,),
      in_specs=[
          pl.BlockSpec((blk, di), lambda i: (i, 0)),
          pl.BlockSpec((di, dh), lambda i: (0, 0)),
          pl.BlockSpec((blk, 1), lambda i: (i, 0)),
      ],
      out_specs=pl.BlockSpec((blk, dh), lambda i: (i, 0)),
      out_shape=jax.ShapeDtypeStruct((n, dh), jnp.float32),
  )(x, W1, dis2)


def _stage2(s1, hp1, dis2, b1, W2, blk):
  n, dh = hp1.shape
  do = W2.shape[1]

  def body(s_ref, hp_ref, dis_ref, b_ref, w_ref, out_ref):
    dis = dis_ref[...]
    z = dis * (s_ref[0] + s_ref[1] + hp_ref[...]) + b_ref[...]
    a = jnp.where(z >= 0, z, 0.01 * z)
    h2 = jnp.dot(a, w_ref[...], precision=lax.Precision.HIGHEST,
                 preferred_element_type=jnp.float32)
    out_ref[...] = h2 * dis

  return pl.pallas_call(
      body,
      grid=(n // blk,),
      in_specs=[
          pl.BlockSpec((NC, blk, dh), lambda i: (0, i, 0)),
          pl.BlockSpec((blk, dh), lambda i: (i, 0)),
          pl.BlockSpec((blk, 1), lambda i: (i, 0)),
          pl.BlockSpec((1, dh), lambda i: (0, 0)),
          pl.BlockSpec((dh, do), lambda i: (0, 0)),
      ],
      out_specs=pl.BlockSpec((blk, do), lambda i: (i, 0)),
      out_shape=jax.ShapeDtypeStruct((n, do), jnp.float32),
  )(s1, hp1, dis2, b1, W2)


def _stage3(s2, hp2, dis2, b2, gamma, beta, blk):
  n, do = hp2.shape

  def body(s_ref, hp_ref, dis_ref, b_ref, g_ref, be_ref, out_ref):
    z = dis_ref[...] * (s_ref[0] + s_ref[1] + hp_ref[...]) + b_ref[...]
    mu = jnp.mean(z, axis=-1, keepdims=True)
    var = jnp.mean((z - mu) ** 2, axis=-1, keepdims=True)
    out_ref[...] = (z - mu) * lax.rsqrt(var + 1e-5) * g_ref[...] + be_ref[...]

  return pl.pallas_call(
      body,
      grid=(n // blk,),
      in_specs=[
          pl.BlockSpec((NC, blk, do), lambda i: (0, i, 0)),
          pl.BlockSpec((blk, do), lambda i: (i, 0)),
          pl.BlockSpec((blk, 1), lambda i: (i, 0)),
          pl.BlockSpec((1, do), lambda i: (0, 0)),
          pl.BlockSpec((1, do), lambda i: (0, 0)),
          pl.BlockSpec((1, do), lambda i: (0, 0)),
      ],
      out_specs=pl.BlockSpec((blk, do), lambda i: (i, 0)),
      out_shape=jax.ShapeDtypeStruct((n, do), jnp.float32),
  )(s2, hp2, dis2, b2, gamma, beta)


def kernel(x, edge_index, edge_weight, W1, b1, W2, b2, gamma, beta):
  n = x.shape[0]
  e = edge_weight.shape[0]
  src, dst = edge_index[0], edge_index[1]

  kc = -(-e // (NW * SUB))
  kc = -(-kc // PB) * PB
  pad = NW * kc * SUB - e
  src3 = jnp.pad(src, (0, pad)).reshape(NW, kc, SUB)
  dst3 = jnp.pad(dst, (0, pad)).reshape(NW, kc, SUB)
  ew3 = jnp.pad(edge_weight, (0, pad)).reshape(NW, kc, SUB)

  blk = 1000

  degp = _deg_call(n, kc, dst3, ew3)
  dis2 = _stage0(degp)
  hp1 = _stage1(x, W1, dis2, blk)
  s1 = _msg_call(n, hp1.shape[1], kc, hp1, src3, dst3, ew3)
  hp2 = _stage2(s1, hp1, dis2, b1.reshape(1, -1), W2, blk)
  s2 = _msg_call(n, hp2.shape[1], kc, hp2, src3, dst3, ew3)
  return _stage3(s2, hp2, dis2, b2.reshape(1, -1),
                 gamma.reshape(1, -1), beta.reshape(1, -1), blk)


# speedup vs baseline: 1.4167x; 1.2063x over previous
import functools

import jax
import jax.numpy as jnp
from jax import lax
from jax.experimental import pallas as pl
from jax.experimental.pallas import tpu as pltpu
from jax.experimental.pallas import tpu_sc as plsc

NC = 2
NS = 16
NW = NC * NS
L = 16
SUB = 128


def _wid():
  return lax.axis_index("s") * NC + lax.axis_index("c")


def _deg_body(n, kc, dst_hbm, ew_hbm, out_hbm, dst_v, ew_v, deg_v):
  wid = _wid()
  pltpu.sync_copy(dst_hbm.at[wid], dst_v)
  pltpu.sync_copy(ew_hbm.at[wid], ew_v)

  zero = jnp.zeros((L,), jnp.float32)

  def zbody(i, _):
    deg_v[pl.ds(i * L, L)] = zero
    return 0

  lax.fori_loop(0, n // L, zbody, 0)

  def ebody(i, _):
    j = i // (SUB // L)
    g = i % (SUB // L)
    idx = dst_v[j, pl.ds(g * L, L)]
    w = ew_v[j, pl.ds(g * L, L)]
    plsc.addupdate_scatter(deg_v, [idx], w)
    return 0

  lax.fori_loop(0, kc * (SUB // L), ebody, 0)
  pltpu.sync_copy(deg_v, out_hbm.at[wid])


def _deg_call(n, kc, dst3, ew3):
  mesh = plsc.VectorSubcoreMesh(core_axis_name="c", subcore_axis_name="s")
  return pl.kernel(
      functools.partial(_deg_body, n, kc),
      out_type=jax.ShapeDtypeStruct((NW, n), jnp.float32),
      mesh=mesh,
      compiler_params=pltpu.CompilerParams(needs_layout_passes=False),
      scratch_types=[
          pltpu.VMEM((kc, SUB), jnp.int32),
          pltpu.VMEM((kc, SUB), jnp.float32),
          pltpu.VMEM((n,), jnp.float32),
      ],
  )(dst3, ew3)


def _msg_body(n, d, kc, h_hbm, src_hbm, dst_hbm, ew_hbm, out_hbm,
              src_v, dst_v, ew_v, rows_v, acc_sp):
  c = lax.axis_index("c")
  s = lax.axis_index("s")
  wid = s * NC + c
  nv = d // L

  pltpu.sync_copy(src_hbm.at[wid], src_v)
  pltpu.sync_copy(dst_hbm.at[wid], dst_v)
  pltpu.sync_copy(ew_hbm.at[wid], ew_v)

  zero = jnp.zeros((L,), jnp.float32)

  def zbody(i, _):
    rows_v[i // nv, pl.ds((i % nv) * L, L)] = zero
    return 0

  lax.fori_loop(0, SUB * nv, zbody, 0)
  stride = (n // NS) // 8 * 8
  nwin = -(-(n - stride * (NS - 1)) // SUB)
  base = pl.multiple_of(s * stride, 8)
  for t in range(nwin):
    pltpu.sync_copy(rows_v, acc_sp.at[pl.ds(base + t * SUB, SUB)])
  plsc.subcore_barrier()

  lane_ids = [jnp.full((L,), l, jnp.int32) for l in range(L)]

  def chunk(j, _):
    pltpu.sync_copy(h_hbm.at[src_v.at[j]], rows_v)

    def grp(g, _):
      ewv = ew_v[j, pl.ds(g * L, L)]
      for l in range(L):
        b = jnp.take_along_axis(ewv, lane_ids[l], axis=0)
        e = g * L + l
        for k in range(nv):
          sl = pl.ds(k * L, L)
          rows_v[e, sl] = rows_v[e, sl] * b
      return 0

    lax.fori_loop(0, SUB // L, grp, 0)

    pltpu.sync_copy(rows_v, acc_sp.at[dst_v.at[j]], add=True)
    return 0

  lax.fori_loop(0, kc, chunk, 0)
  plsc.subcore_barrier()
  pltpu.sync_copy(acc_sp.at[pl.ds(base, nwin * SUB)],
                  out_hbm.at[c, pl.ds(base, nwin * SUB)])


def _msg_call(n, d, kc, h, src3, dst3, ew3):
  mesh = plsc.VectorSubcoreMesh(core_axis_name="c", subcore_axis_name="s")
  return pl.kernel(
      functools.partial(_msg_body, n, d, kc),
      out_type=jax.ShapeDtypeStruct((NC, n, d), jnp.float32),
      mesh=mesh,
      compiler_params=pltpu.CompilerParams(needs_layout_passes=False),
      scratch_types=[
          pltpu.VMEM((kc, SUB), jnp.int32),
          pltpu.VMEM((kc, SUB), jnp.int32),
          pltpu.VMEM((kc, SUB), jnp.float32),
          pltpu.VMEM((SUB, d), jnp.float32),
          pltpu.VMEM_SHARED((n, d), jnp.float32),
      ],
  )(h, src3, dst3, ew3)


def _stage0(degp):
  n = degp.shape[1]

  def body(degp_ref, out_ref):
    deg = jnp.sum(degp_ref[...], axis=0) + 1.0
    pos = deg > 0
    dis = jnp.where(pos, lax.rsqrt(jnp.where(pos, deg, 1.0)), 0.0)
    out_ref[...] = dis[:, None]

  return pl.pallas_call(
      body,
      out_shape=jax.ShapeDtypeStruct((n, 1), jnp.float32),
  )(degp)


def _stage1(x, W1, dis2, blk):
  n, di = x.shape
  dh = W1.shape[1]

  def body(x_ref, w_ref, dis_ref, out_ref):
    h = jnp.dot(x_ref[...], w_ref[...], precision=lax.Precision.HIGHEST,
                preferred_element_type=jnp.float32)
    out_ref[...] = h * dis_ref[...]

  return pl.pallas_call(
      body,
      grid=(n // blk,),
      in_specs=[
          pl.BlockSpec((blk, di), lambda i: (i, 0)),
          pl.BlockSpec((di, dh), lambda i: (0, 0)),
          pl.BlockSpec((blk, 1), lambda i: (i, 0)),
      ],
      out_specs=pl.BlockSpec((blk, dh), lambda i: (i, 0)),
      out_shape=jax.ShapeDtypeStruct((n, dh), jnp.float32),
  )(x, W1, dis2)


def _stage2(s1, hp1, dis2, b1, W2, blk):
  n, dh = hp1.shape
  do = W2.shape[1]

  def body(s_ref, hp_ref, dis_ref, b_ref, w_ref, out_ref):
    dis = dis_ref[...]
    z = dis * (s_ref[0] + s_ref[1] + hp_ref[...]) + b_ref[...]
    a = jnp.where(z >= 0, z, 0.01 * z)
    h2 = jnp.dot(a, w_ref[...], precision=lax.Precision.HIGHEST,
                 preferred_element_type=jnp.float32)
    out_ref[...] = h2 * dis

  return pl.pallas_call(
      body,
      grid=(n // blk,),
      in_specs=[
          pl.BlockSpec((NC, blk, dh), lambda i: (0, i, 0)),
          pl.BlockSpec((blk, dh), lambda i: (i, 0)),
          pl.BlockSpec((blk, 1), lambda i: (i, 0)),
          pl.BlockSpec((1, dh), lambda i: (0, 0)),
          pl.BlockSpec((dh, do), lambda i: (0, 0)),
      ],
      out_specs=pl.BlockSpec((blk, do), lambda i: (i, 0)),
      out_shape=jax.ShapeDtypeStruct((n, do), jnp.float32),
  )(s1, hp1, dis2, b1, W2)


def _stage3(s2, hp2, dis2, b2, gamma, beta, blk):
  n, do = hp2.shape

  def body(s_ref, hp_ref, dis_ref, b_ref, g_ref, be_ref, out_ref):
    z = dis_ref[...] * (s_ref[0] + s_ref[1] + hp_ref[...]) + b_ref[...]
    mu = jnp.mean(z, axis=-1, keepdims=True)
    var = jnp.mean((z - mu) ** 2, axis=-1, keepdims=True)
    out_ref[...] = (z - mu) * lax.rsqrt(var + 1e-5) * g_ref[...] + be_ref[...]

  return pl.pallas_call(
      body,
      grid=(n // blk,),
      in_specs=[
          pl.BlockSpec((NC, blk, do), lambda i: (0, i, 0)),
          pl.BlockSpec((blk, do), lambda i: (i, 0)),
          pl.BlockSpec((blk, 1), lambda i: (i, 0)),
          pl.BlockSpec((1, do), lambda i: (0, 0)),
          pl.BlockSpec((1, do), lambda i: (0, 0)),
          pl.BlockSpec((1, do), lambda i: (0, 0)),
      ],
      out_specs=pl.BlockSpec((blk, do), lambda i: (i, 0)),
      out_shape=jax.ShapeDtypeStruct((n, do), jnp.float32),
  )(s2, hp2, dis2, b2, gamma, beta)


def kernel(x, edge_index, edge_weight, W1, b1, W2, b2, gamma, beta):
  n = x.shape[0]
  e = edge_weight.shape[0]
  src, dst = edge_index[0], edge_index[1]

  kc = -(-e // (NW * SUB))
  ep = NW * kc * SUB
  pad = ep - e
  src3 = jnp.pad(src, (0, pad)).reshape(NW, kc, SUB)
  dst3 = jnp.pad(dst, (0, pad)).reshape(NW, kc, SUB)
  ew3 = jnp.pad(edge_weight, (0, pad)).reshape(NW, kc, SUB)

  blk = 1000

  degp = _deg_call(n, kc, dst3, ew3)
  dis2 = _stage0(degp)
  hp1 = _stage1(x, W1, dis2, blk)
  s1 = _msg_call(n, hp1.shape[1], kc, hp1, src3, dst3, ew3)
  hp2 = _stage2(s1, hp1, dis2, b1.reshape(1, -1), W2, blk)
  s2 = _msg_call(n, hp2.shape[1], kc, hp2, src3, dst3, ew3)
  return _stage3(s2, hp2, dis2, b2.reshape(1, -1),
                 gamma.reshape(1, -1), beta.reshape(1, -1), blk)
